# Initial kernel scaffold; baseline (speedup 1.0000x reference)
import jax
import jax.numpy as jnp
from jax.experimental import pallas as pl


def kernel(x, edge_index, agg_alpha, W, bias):
    raise NotImplementedError("write your pallas kernel here")


# speedup vs baseline: 47.9941x; 47.9941x over previous
import functools

import jax
import jax.numpy as jnp
from jax import lax
from jax.experimental import pallas as pl
from jax.experimental.pallas import tpu as pltpu
from jax.experimental.pallas import tpu_sc as plsc

N = 10000
E = 320000
HEADS = 8
DIM = 16
D = HEADS * DIM

NC = 2
NS = 16
NW = NC * NS

C = 128
PER_TILE = 10112
NCHUNK = PER_TILE // C
E_PAD = PER_TILE * NW

STRIPE = 632
STRIPE_LAST = N - 15 * STRIPE


def _mm_body(x_ref, w_ref, o_ref):
    o_ref[...] = jnp.dot(x_ref[...], w_ref[...],
                         preferred_element_type=jnp.float32)


def _project(x, W):
    return pl.pallas_call(
        _mm_body,
        grid=(10,),
        in_specs=[
            pl.BlockSpec((N // 10, D), lambda i: (i, 0)),
            pl.BlockSpec((D, D), lambda i: (0, 0)),
        ],
        out_specs=pl.BlockSpec((N // 10, D), lambda i: (i, 0)),
        out_shape=jax.ShapeDtypeStruct((N, D), jnp.float32),
    )(x, W)


def _combine2(partials, bias):
    bias2d = jnp.broadcast_to(bias.reshape(1, D), (8, D))

    def body(p_ref, b_ref, o_ref):
        o_ref[...] = p_ref[0] + p_ref[1] + b_ref[0:1, :]

    return pl.pallas_call(
        body,
        grid=(10,),
        in_specs=[
            pl.BlockSpec((2, N // 10, D), lambda i: (0, i, 0)),
            pl.BlockSpec((8, D), lambda i: (0, 0)),
        ],
        out_specs=pl.BlockSpec((N // 10, D), lambda i: (i, 0)),
        out_shape=jax.ShapeDtypeStruct((N, D), jnp.float32),
    )(partials, bias2d)


def _sc_body(h_hbm, src_hbm, dst_hbm, alpha_hbm, zeros_hbm, out_hbm,
             srcidx, dstidx, alphav, rows, acc, gsem):
    c_id = lax.axis_index("c")
    s_id = lax.axis_index("s")
    wid = c_id * NS + s_id

    start = pl.multiple_of(s_id * STRIPE, 8)

    @pl.when(s_id < NS - 1)
    def _zero_main():
        sl = pl.ds(start, STRIPE)
        pltpu.sync_copy(zeros_hbm.at[sl], acc.at[sl])

    @pl.when(s_id == NS - 1)
    def _zero_last():
        sl = pl.ds((NS - 1) * STRIPE, STRIPE_LAST)
        pltpu.sync_copy(zeros_hbm.at[sl], acc.at[sl])

    plsc.subcore_barrier()

    base0 = wid * PER_TILE

    def chunk(j, carry):
        base = base0 + j * C
        pltpu.sync_copy(src_hbm.at[pl.ds(base, C)], srcidx)
        pltpu.sync_copy(dst_hbm.at[pl.ds(base, C)], dstidx)
        pltpu.sync_copy(alpha_hbm.at[pl.ds(base * HEADS, C * HEADS)],
                        alphav.at[pl.ds(0, C * HEADS)])
        pltpu.async_copy(h_hbm.at[srcidx], rows, gsem).wait()

        def edge(c, carry2):
            av = alphav[pl.ds(c * HEADS, 16)]
            for hd in range(HEADS):
                rows[c, pl.ds(hd * DIM, DIM)] = (
                    rows[c, pl.ds(hd * DIM, DIM)] * av[hd])
            return carry2

        lax.fori_loop(0, C, edge, 0, unroll=2)
        pltpu.sync_copy(rows, acc.at[dstidx], add=True)
        return carry

    lax.fori_loop(0, NCHUNK, chunk, 0)
    plsc.subcore_barrier()

    @pl.when(s_id < NS - 1)
    def _flush_main():
        sl = pl.ds(start, STRIPE)
        pltpu.sync_copy(acc.at[sl], out_hbm.at[c_id].at[sl])

    @pl.when(s_id == NS - 1)
    def _flush_last():
        sl = pl.ds((NS - 1) * STRIPE, STRIPE_LAST)
        pltpu.sync_copy(acc.at[sl], out_hbm.at[c_id].at[sl])


@functools.partial(jax.jit, static_argnums=())
def _sc_scatter(h, src, dst, alpha, zeros):
    mesh = plsc.VectorSubcoreMesh(core_axis_name="c", subcore_axis_name="s",
                                  num_cores=NC, num_subcores=NS)
    f = pl.kernel(
        _sc_body,
        out_type=jax.ShapeDtypeStruct((NC, N, D), jnp.float32),
        mesh=mesh,
        scratch_types=[
            pltpu.VMEM((C,), jnp.int32),
            pltpu.VMEM((C,), jnp.int32),
            pltpu.VMEM((C * HEADS + 16,), jnp.float32),
            pltpu.VMEM((C, D), jnp.float32),
            pltpu.VMEM_SHARED((N, D), jnp.float32),
            pltpu.SemaphoreType.DMA,
        ],
    )
    return f(h, src, dst, alpha, zeros)


def kernel(x, edge_index, agg_alpha, W, bias):
    h = _project(x, W)
    pad = E_PAD - E
    src = jnp.concatenate([edge_index[0], jnp.zeros((pad,), jnp.int32)])
    dst = jnp.concatenate([edge_index[1], jnp.zeros((pad,), jnp.int32)])
    alpha = jnp.concatenate(
        [agg_alpha, jnp.zeros((pad, HEADS), jnp.float32)], axis=0).reshape(-1)
    zeros = jnp.zeros((N, D), jnp.float32)
    partials = _sc_scatter(h, src, dst, alpha, zeros)
    return _combine2(partials, bias)


# speedup vs baseline: 49.4184x; 1.0297x over previous
import functools

import jax
import jax.numpy as jnp
from jax import lax
from jax.experimental import pallas as pl
from jax.experimental.pallas import tpu as pltpu
from jax.experimental.pallas import tpu_sc as plsc

N = 10000
E = 320000
HEADS = 8
DIM = 16
D = HEADS * DIM

NC = 2
NS = 16
NW = NC * NS

C = 64
NBUF = 4
NCHUNK = 160
PER_TILE = NCHUNK * C
E_PAD = PER_TILE * NW

STRIPE = 632
STRIPE_LAST = N - 15 * STRIPE


def _mm_body(x_ref, w_ref, o_ref):
    o_ref[...] = jnp.dot(x_ref[...], w_ref[...],
                         preferred_element_type=jnp.float32)


def _project(x, W):
    return pl.pallas_call(
        _mm_body,
        grid=(10,),
        in_specs=[
            pl.BlockSpec((N // 10, D), lambda i: (i, 0)),
            pl.BlockSpec((D, D), lambda i: (0, 0)),
        ],
        out_specs=pl.BlockSpec((N // 10, D), lambda i: (i, 0)),
        out_shape=jax.ShapeDtypeStruct((N, D), jnp.float32),
    )(x, W)


def _combine2(partials, bias):
    bias2d = jnp.broadcast_to(bias.reshape(1, D), (8, D))

    def body(p_ref, b_ref, o_ref):
        o_ref[...] = p_ref[0] + p_ref[1] + b_ref[0:1, :]

    return pl.pallas_call(
        body,
        grid=(10,),
        in_specs=[
            pl.BlockSpec((2, N // 10, D), lambda i: (0, i, 0)),
            pl.BlockSpec((8, D), lambda i: (0, 0)),
        ],
        out_specs=pl.BlockSpec((N // 10, D), lambda i: (i, 0)),
        out_shape=jax.ShapeDtypeStruct((N, D), jnp.float32),
    )(partials, bias2d)


def _sc_body(h_hbm, src_hbm, dst_hbm, alpha_hbm, zeros_hbm, out_hbm,
             srcidx, dstidx, alphav, rows, acc, gsem, ssem, isem):
    c_id = lax.axis_index("c")
    s_id = lax.axis_index("s")
    wid = c_id * NS + s_id

    start = pl.multiple_of(s_id * STRIPE, 8)

    @pl.when(s_id < NS - 1)
    def _zero_main():
        sl = pl.ds(start, STRIPE)
        pltpu.sync_copy(zeros_hbm.at[sl], acc.at[sl])

    @pl.when(s_id == NS - 1)
    def _zero_last():
        sl = pl.ds((NS - 1) * STRIPE, STRIPE_LAST)
        pltpu.sync_copy(zeros_hbm.at[sl], acc.at[sl])

    plsc.subcore_barrier()

    base0 = wid * PER_TILE


    def i_descs(j, p):
        base = base0 + j * C
        return (
            pltpu.make_async_copy(
                src_hbm.at[pl.ds(base, C)], srcidx.at[p], isem.at[p]),
            pltpu.make_async_copy(
                dst_hbm.at[pl.ds(base, C)], dstidx.at[p], isem.at[p]),
            pltpu.make_async_copy(
                alpha_hbm.at[pl.ds(base * HEADS, C * HEADS)],
                alphav.at[p], isem.at[p]),
        )

    def g_desc(p):
        return pltpu.make_async_copy(
            h_hbm.at[srcidx.at[p]], rows.at[p], gsem.at[p])

    def s_desc(p):
        return pltpu.make_async_copy(
            rows.at[p], acc.at[dstidx.at[p]], ssem.at[p])

    def multiply(p):
        def pair(q, carry2):
            av = alphav[p, pl.ds(q * 16, 16)]
            for e in range(2):
                c = 2 * q + e
                for hd in range(HEADS):
                    rows[p, c, pl.ds(hd * DIM, DIM)] = (
                        rows[p, c, pl.ds(hd * DIM, DIM)] * av[8 * e + hd])
            return carry2

        lax.fori_loop(0, C // 2, pair, 0, unroll=2)

    for d in i_descs(0, 0):
        d.start()
    for d in i_descs(1, 1):
        d.start()
    for d in i_descs(0, 0):
        d.wait()
    g_desc(0).start()

    def body(t, carry):
        for p in range(NBUF):
            j = NBUF * t + p
            p1 = (p + 1) % NBUF
            p2 = (p + 2) % NBUF

            @pl.when(j >= 2)
            def _wait_s():
                s_desc(p2).wait()

            @pl.when(j + 2 < NCHUNK)
            def _issue_i():
                for d in i_descs(j + 2, p2):
                    d.start()

            @pl.when(j + 1 < NCHUNK)
            def _issue_g():
                for d in i_descs(j + 1, p1):
                    d.wait()
                g_desc(p1).start()

            g_desc(p).wait()
            multiply(p)
            pltpu.async_copy(
                rows.at[p], acc.at[dstidx.at[p]], ssem.at[p], add=True)
        return carry

    lax.fori_loop(0, NCHUNK // NBUF, body, 0)
    for p in (NBUF - 2, NBUF - 1):
        s_desc(p).wait()
    plsc.subcore_barrier()

    @pl.when(s_id < NS - 1)
    def _flush_main():
        sl = pl.ds(start, STRIPE)
        pltpu.sync_copy(acc.at[sl], out_hbm.at[c_id].at[sl])

    @pl.when(s_id == NS - 1)
    def _flush_last():
        sl = pl.ds((NS - 1) * STRIPE, STRIPE_LAST)
        pltpu.sync_copy(acc.at[sl], out_hbm.at[c_id].at[sl])


@functools.partial(jax.jit, static_argnums=())
def _sc_scatter(h, src, dst, alpha, zeros):
    mesh = plsc.VectorSubcoreMesh(core_axis_name="c", subcore_axis_name="s",
                                  num_cores=NC, num_subcores=NS)
    f = pl.kernel(
        _sc_body,
        out_type=jax.ShapeDtypeStruct((NC, N, D), jnp.float32),
        mesh=mesh,
        scratch_types=[
            pltpu.VMEM((NBUF, C), jnp.int32),
            pltpu.VMEM((NBUF, C), jnp.int32),
            pltpu.VMEM((NBUF, C * HEADS), jnp.float32),
            pltpu.VMEM((NBUF, C, D), jnp.float32),
            pltpu.VMEM_SHARED((N, D), jnp.float32),
            pltpu.SemaphoreType.DMA((NBUF,)),
            pltpu.SemaphoreType.DMA((NBUF,)),
            pltpu.SemaphoreType.DMA((NBUF,)),
        ],
    )
    return f(h, src, dst, alpha, zeros)


def kernel(x, edge_index, agg_alpha, W, bias):
    h = _project(x, W)
    pad = E_PAD - E
    src = jnp.concatenate([edge_index[0], jnp.zeros((pad,), jnp.int32)])
    dst = jnp.concatenate([edge_index[1], jnp.zeros((pad,), jnp.int32)])
    alpha = jnp.concatenate(
        [agg_alpha, jnp.zeros((pad, HEADS), jnp.float32)], axis=0).reshape(-1)
    zeros = jnp.zeros((N, D), jnp.float32)
    partials = _sc_scatter(h, src, dst, alpha, zeros)
    return _combine2(partials, bias)


# speedup vs baseline: 56.9195x; 1.1518x over previous
import functools

import jax
import jax.numpy as jnp
from jax import lax
from jax.experimental import pallas as pl
from jax.experimental.pallas import tpu as pltpu
from jax.experimental.pallas import tpu_sc as plsc

N = 10000
E = 320000
HEADS = 8
DIM = 16
D = HEADS * DIM

NC = 2
NS = 16
NW = NC * NS

C = 64
NBUF = 4
NCHUNK = 160
PER_TILE = NCHUNK * C
E_PAD = PER_TILE * NW
NMETA = 2 + HEADS

STRIPE = 632
STRIPE_LAST = N - 15 * STRIPE


def _mm_body(x_ref, w_ref, o_ref):
    o_ref[...] = jnp.dot(x_ref[...], w_ref[...],
                         preferred_element_type=jnp.float32)


def _project(x, W):
    return pl.pallas_call(
        _mm_body,
        grid=(10,),
        in_specs=[
            pl.BlockSpec((N // 10, D), lambda i: (i, 0)),
            pl.BlockSpec((D, D), lambda i: (0, 0)),
        ],
        out_specs=pl.BlockSpec((N // 10, D), lambda i: (i, 0)),
        out_shape=jax.ShapeDtypeStruct((N, D), jnp.float32),
    )(x, W)


def _combine2(partials, bias):
    bias2d = jnp.broadcast_to(bias.reshape(1, D), (8, D))

    def body(p_ref, b_ref, o_ref):
        o_ref[...] = p_ref[0] + p_ref[1] + b_ref[0:1, :]

    return pl.pallas_call(
        body,
        grid=(10,),
        in_specs=[
            pl.BlockSpec((2, N // 10, D), lambda i: (0, i, 0)),
            pl.BlockSpec((8, D), lambda i: (0, 0)),
        ],
        out_specs=pl.BlockSpec((N // 10, D), lambda i: (i, 0)),
        out_shape=jax.ShapeDtypeStruct((N, D), jnp.float32),
    )(partials, bias2d)


def _sc_body(h_hbm, meta_hbm, zeros_hbm, out_hbm,
             meta, rows, acc, gsem, ssem, isem):
    c_id = lax.axis_index("c")
    s_id = lax.axis_index("s")
    wid = c_id * NS + s_id

    start = pl.multiple_of(s_id * STRIPE, 8)

    @pl.when(s_id < NS - 1)
    def _zero_main():
        sl = pl.ds(start, STRIPE)
        pltpu.sync_copy(zeros_hbm.at[sl], acc.at[sl])

    @pl.when(s_id == NS - 1)
    def _zero_last():
        sl = pl.ds((NS - 1) * STRIPE, STRIPE_LAST)
        pltpu.sync_copy(zeros_hbm.at[sl], acc.at[sl])

    plsc.subcore_barrier()

    chunk0 = wid * NCHUNK


    def i_desc(j, p):
        return pltpu.make_async_copy(
            meta_hbm.at[chunk0 + j], meta.at[p], isem.at[p])

    def g_desc(p):
        return pltpu.make_async_copy(
            h_hbm.at[meta.at[p, 0]], rows.at[p], gsem.at[p])

    def s_desc(p):
        return pltpu.make_async_copy(
            rows.at[p], acc.at[meta.at[p, 1]], ssem.at[p])

    def multiply(p):
        def pair(q, carry2):
            r = 2 + q // (C // 16)
            col = 16 * (q % (C // 16))
            av = lax.bitcast_convert_type(
                meta[p, r, pl.ds(col, 16)], jnp.float32)
            for e in range(2):
                c = 2 * q + e
                for hd in range(HEADS):
                    rows[p, c, pl.ds(hd * DIM, DIM)] = (
                        rows[p, c, pl.ds(hd * DIM, DIM)] * av[8 * e + hd])
            return carry2

        lax.fori_loop(0, C // 2, pair, 0, unroll=2)

    for jj in range(2):
        i_desc(jj, jj).start()
    for jj in range(2):
        i_desc(jj, jj).wait()
        g_desc(jj).start()

    def body(t, carry):
        for p in range(NBUF):
            j = NBUF * t + p
            p2 = (p + 2) % NBUF

            @pl.when(j >= 2)
            def _wait_s():
                s_desc(p2).wait()

            @pl.when(j + 2 < NCHUNK)
            def _issue_i():
                i_desc(j + 2, p2).start()

            g_desc(p).wait()
            multiply(p)
            pltpu.async_copy(
                rows.at[p], acc.at[meta.at[p, 1]], ssem.at[p], add=True)

            @pl.when(j + 2 < NCHUNK)
            def _issue_g():
                i_desc(j + 2, p2).wait()
                g_desc(p2).start()
        return carry

    lax.fori_loop(0, NCHUNK // NBUF, body, 0)
    for p in ((NCHUNK - 2) % NBUF, (NCHUNK - 1) % NBUF):
        s_desc(p).wait()
    plsc.subcore_barrier()

    @pl.when(s_id < NS - 1)
    def _flush_main():
        sl = pl.ds(start, STRIPE)
        pltpu.sync_copy(acc.at[sl], out_hbm.at[c_id].at[sl])

    @pl.when(s_id == NS - 1)
    def _flush_last():
        sl = pl.ds((NS - 1) * STRIPE, STRIPE_LAST)
        pltpu.sync_copy(acc.at[sl], out_hbm.at[c_id].at[sl])


@functools.partial(jax.jit, static_argnums=())
def _sc_scatter(h, meta, zeros):
    mesh = plsc.VectorSubcoreMesh(core_axis_name="c", subcore_axis_name="s",
                                  num_cores=NC, num_subcores=NS)
    f = pl.kernel(
        _sc_body,
        out_type=jax.ShapeDtypeStruct((NC, N, D), jnp.float32),
        mesh=mesh,
        scratch_types=[
            pltpu.VMEM((NBUF, NMETA, C), jnp.int32),
            pltpu.VMEM((NBUF, C, D), jnp.float32),
            pltpu.VMEM_SHARED((N, D), jnp.float32),
            pltpu.SemaphoreType.DMA((NBUF,)),
            pltpu.SemaphoreType.DMA((NBUF,)),
            pltpu.SemaphoreType.DMA((NBUF,)),
        ],
    )
    return f(h, meta, zeros)


def kernel(x, edge_index, agg_alpha, W, bias):
    h = _project(x, W)
    pad = E_PAD - E
    tot = E_PAD // C
    src = jnp.concatenate([edge_index[0], jnp.zeros((pad,), jnp.int32)])
    dst = jnp.concatenate([edge_index[1], jnp.zeros((pad,), jnp.int32)])
    alpha = jnp.concatenate(
        [agg_alpha, jnp.zeros((pad, HEADS), jnp.float32)], axis=0)
    meta = jnp.concatenate(
        [src.reshape(tot, 1, C),
         dst.reshape(tot, 1, C),
         lax.bitcast_convert_type(alpha, jnp.int32).reshape(tot, HEADS, C)],
        axis=1)
    zeros = jnp.zeros((N, D), jnp.float32)
    partials = _sc_scatter(h, meta, zeros)
    return _combine2(partials, bias)
